# baseline (speedup 1.0000x reference)
import functools

import jax
import numpy as np
import jax.numpy as jnp
from jax import lax
from jax.experimental import pallas as pl
from jax.experimental.pallas import tpu as pltpu
from jax.experimental.pallas import tpu_sc as plsc

B, N, C = 8, 16128, 85
K_NUM = 20
NCORE, NSUB, L = 2, 16, 16
NCHUNK = 4
R = N // NCHUNK
GROUP = 64
G = R // GROUP
G1 = 32
R1 = G1 * GROUP
NEG = np.float32(-3.0e38)

_f32 = jnp.float32
_i32 = jnp.int32


def _iota16():
    return lax.iota(_i32, L)


def _find_max_and_group(gmax_ref):
    vs = [gmax_ref[pl.ds(q * L, L)] for q in range(4)]
    m = jnp.max(jnp.maximum(jnp.maximum(vs[0], vs[1]),
                            jnp.maximum(vs[2], vs[3])))
    iota = _iota16()
    big = np.int32(4 * L)
    cand = jnp.minimum(
        jnp.minimum(jnp.where(vs[0] == m, iota, big),
                    jnp.where(vs[1] == m, iota + L, big)),
        jnp.minimum(jnp.where(vs[2] == m, iota + 2 * L, big),
                    jnp.where(vs[3] == m, iota + 3 * L, big)))
    return m, jnp.min(cand)


def _remove_one_and_refresh(valbuf, gmax_ref, g, m):
    iota = _iota16()
    base = g * GROUP
    vs = [valbuf[pl.ds(base + q * L, L)] for q in range(4)]
    big = np.int32(GROUP)
    pos = jnp.min(jnp.minimum(
        jnp.minimum(jnp.where(vs[0] == m, iota, big),
                    jnp.where(vs[1] == m, iota + L, big)),
        jnp.minimum(jnp.where(vs[2] == m, iota + 2 * L, big),
                    jnp.where(vs[3] == m, iota + 3 * L, big))))
    gm = NEG
    for q in range(4):
        v2 = jnp.where(iota + q * L == pos, NEG, vs[q])
        valbuf[pl.ds(base + q * L, L)] = v2
        gm = jnp.maximum(gm, jnp.max(v2))
    plsc.store_scatter(gmax_ref, [jnp.full((L,), g, _i32)],
                       jnp.full((L,), gm, _f32), mask=iota == 0)


def _extract_topk(valbuf, gmax_ref, top_ref):
    def body(i, acc):
        m, g = _find_max_and_group(gmax_ref)
        _remove_one_and_refresh(valbuf, gmax_ref, g, m)
        plsc.store_scatter(top_ref, [jnp.full((L,), i, _i32)],
                           jnp.full((L,), m, _f32), mask=_iota16() == 0)
        return acc + m
    return lax.fori_loop(0, K_NUM, body, np.float32(0.0))


def _merge_topk_sum(merge_ref):
    iota = _iota16()
    gm0 = jnp.full((L,), NEG, _f32)
    for j in range(8):
        gm0 = jnp.where(iota == j, jnp.max(merge_ref[pl.ds(j * L, L)]), gm0)

    def body(i, carry):
        acc, gm = carry
        m = jnp.max(gm)
        g = jnp.min(jnp.where(gm == m, iota, L))
        v = merge_ref[pl.ds(g * L, L)]
        pos = jnp.min(jnp.where(v == m, iota, L))
        v2 = jnp.where(iota == pos, NEG, v)
        merge_ref[pl.ds(g * L, L)] = v2
        gm = jnp.where(iota == g, jnp.max(v2), gm)
        return (acc + m, gm)

    acc, _ = lax.fori_loop(0, K_NUM, body, (np.float32(0.0), gm0))
    return acc


def _sc_body(y_hbm, gt_hbm, out_hbm,
             cbx, cby, cbw, cbh, cbo, cbc, ioub, confb, gmax_i, gmax_c, gtv,
             top_i, top_c, shared_i, shared_c, merge_i, merge_c, outv,
             sem0, sem1, sem2, sem3, sem4, sem5,
             sem6, sem7, sem8, sem9, sem10, sem11):
    c = lax.axis_index("c")
    s = lax.axis_index("s")
    wid = c * NSUB + s
    b = wid // NCHUNK
    chunk = wid % NCHUNK

    row_lo = chunk * R
    sems_a = (sem0, sem1, sem2, sem3, sem4, sem5)
    sems_b = (sem6, sem7, sem8, sem9, sem10, sem11)
    bufs = (cbx, cby, cbw, cbh, cbo, cbc)
    copies_a = [
        pltpu.async_copy(y_hbm.at[b, cc, pl.ds(row_lo, R1)],
                         cbuf.at[pl.ds(0, R1)], sems_a[cc])
        for cc, cbuf in enumerate(bufs)
    ]
    copies_b = [
        pltpu.async_copy(y_hbm.at[b, cc, pl.ds(row_lo + R1, R - R1)],
                         cbuf.at[pl.ds(R1, R - R1)], sems_b[cc])
        for cc, cbuf in enumerate(bufs)
    ]

    pltpu.sync_copy(gt_hbm.at[b], gtv)

    gv = gtv[pl.ds(0, L)]
    gx1, gy1, gx2, gy2, garea = gv[0], gv[1], gv[2], gv[3], gv[4]

    neg = jnp.full((L,), NEG, _f32)
    for q in range(4):
        gmax_i[pl.ds(q * L, L)] = neg
        gmax_c[pl.ds(q * L, L)] = neg
    for q in range(2):
        top_i[pl.ds(q * L, L)] = neg
        top_c[pl.ds(q * L, L)] = neg

    iota = _iota16()
    lane0 = iota == 0

    def group_body(g, _):
        gm_i = neg
        gm_c = neg
        for q in range(4):
            row0 = g * GROUP + q * L
            sl = pl.ds(row0, L)
            cx, cy, w, h, obj, cls = (cbx[sl], cby[sl], cbw[sl], cbh[sl],
                                      cbo[sl], cbc[sl])
            x1 = cx - w * 0.5
            x2 = cx + w * 0.5
            yy1 = cy - h * 0.5
            yy2 = cy + h * 0.5
            ix1 = jnp.maximum(gx1, x1)
            iy1 = jnp.maximum(gy1, yy1)
            ix2 = jnp.minimum(gx2, x2)
            iy2 = jnp.minimum(gy2, yy2)
            iw = jnp.maximum(ix2 - ix1 + 1.0, 0.0)
            ih = jnp.maximum(iy2 - iy1 + 1.0, 0.0)
            inter = iw * ih
            b2a = (x2 - x1 + 1.0) * (yy2 - yy1 + 1.0)
            iou = inter / (garea + b2a - inter + 1e-16)
            conf = obj + cls
            ioub[pl.ds(row0, L)] = iou
            confb[pl.ds(row0, L)] = conf
            gm_i = jnp.maximum(gm_i, iou)
            gm_c = jnp.maximum(gm_c, conf)
        gidx = jnp.full((L,), g, _i32)
        plsc.store_scatter(gmax_i, [gidx],
                           jnp.full((L,), jnp.max(gm_i), _f32), mask=lane0)
        plsc.store_scatter(gmax_c, [gidx],
                           jnp.full((L,), jnp.max(gm_c), _f32), mask=lane0)
        return 0

    for cp in copies_a:
        cp.wait()
    lax.fori_loop(0, G1, group_body, 0)
    for cp in copies_b:
        cp.wait()
    lax.fori_loop(G1, G, group_body, 0)

    _extract_topk(ioub, gmax_i, top_i)
    _extract_topk(confb, gmax_c, top_c)

    pltpu.sync_copy(top_i, shared_i.at[pl.ds(s * 2 * L, 2 * L)])
    pltpu.sync_copy(top_c, shared_c.at[pl.ds(s * 2 * L, 2 * L)])
    plsc.subcore_barrier()

    s0 = (s // NCHUNK) * NCHUNK

    @pl.when(s % NCHUNK == 0)
    def _merge_conf():
        pltpu.sync_copy(shared_c.at[pl.ds(s0 * 2 * L, NCHUNK * 2 * L)],
                        merge_c)
        conf_sum = _merge_topk_sum(merge_c)
        outv[pl.ds(0, L)] = jnp.where(lane0, jnp.full((L,), conf_sum, _f32),
                                      jnp.zeros((L,), _f32))
        pltpu.sync_copy(outv, out_hbm.at[b, 0])

    @pl.when(s % NCHUNK == 1)
    def _merge_iou():
        pltpu.sync_copy(shared_i.at[pl.ds(s0 * 2 * L, NCHUNK * 2 * L)],
                        merge_i)
        iou_sum = _merge_topk_sum(merge_i)
        outv[pl.ds(0, L)] = jnp.where(lane0, jnp.full((L,), iou_sum, _f32),
                                      jnp.zeros((L,), _f32))
        pltpu.sync_copy(outv, out_hbm.at[b, 1])


@functools.partial(jax.jit, static_argnames=())
def _sc_call(y, gtf):
    mesh = plsc.VectorSubcoreMesh(core_axis_name="c", subcore_axis_name="s",
                                  num_cores=NCORE, num_subcores=NSUB)
    fn = pl.kernel(
        _sc_body,
        out_type=jax.ShapeDtypeStruct((B, 2, L), _f32),
        mesh=mesh,
        compiler_params=pltpu.CompilerParams(use_tc_tiling_on_sc=False,
                                             needs_layout_passes=False),
        scratch_types=[
            pltpu.VMEM((R,), _f32),
            pltpu.VMEM((R,), _f32),
            pltpu.VMEM((R,), _f32),
            pltpu.VMEM((R,), _f32),
            pltpu.VMEM((R,), _f32),
            pltpu.VMEM((R,), _f32),
            pltpu.VMEM((R,), _f32),
            pltpu.VMEM((R,), _f32),
            pltpu.VMEM((GROUP,), _f32),
            pltpu.VMEM((GROUP,), _f32),
            pltpu.VMEM((L,), _f32),
            pltpu.VMEM((2 * L,), _f32),
            pltpu.VMEM((2 * L,), _f32),
            pltpu.VMEM_SHARED((NSUB * 2 * L,), _f32),
            pltpu.VMEM_SHARED((NSUB * 2 * L,), _f32),
            pltpu.VMEM((NCHUNK * 2 * L,), _f32),
            pltpu.VMEM((NCHUNK * 2 * L,), _f32),
            pltpu.VMEM((L,), _f32),
            pltpu.SemaphoreType.DMA,
            pltpu.SemaphoreType.DMA,
            pltpu.SemaphoreType.DMA,
            pltpu.SemaphoreType.DMA,
            pltpu.SemaphoreType.DMA,
            pltpu.SemaphoreType.DMA,
            pltpu.SemaphoreType.DMA,
            pltpu.SemaphoreType.DMA,
            pltpu.SemaphoreType.DMA,
            pltpu.SemaphoreType.DMA,
            pltpu.SemaphoreType.DMA,
            pltpu.SemaphoreType.DMA,
        ],
    )
    return fn(y, gtf)


def kernel(YOLOoutput, conf_thres, gt_boxes):
    del conf_thres
    gt = gt_boxes.astype(_f32)
    x1, y1, x2, y2 = gt[:, 0], gt[:, 1], gt[:, 2], gt[:, 3]
    area = (x2 - x1 + 1.0) * (y2 - y1 + 1.0)
    zeros = jnp.zeros_like(x1)
    gtf = jnp.stack([x1, y1, x2, y2, area] + [zeros] * 11, axis=1)
    yt = jnp.transpose(YOLOoutput[:, :, :6], (0, 2, 1))
    out = _sc_call(yt, gtf)
    max_conf = out[:, 0, 0] / K_NUM
    max_ious = jnp.sum(out[:, 1, 0]) / K_NUM
    return (max_conf, max_ious)


# speedup vs baseline: 1.0106x; 1.0106x over previous
import functools

import jax
import numpy as np
import jax.numpy as jnp
from jax import lax
from jax.experimental import pallas as pl
from jax.experimental.pallas import tpu as pltpu
from jax.experimental.pallas import tpu_sc as plsc

B, N, C = 8, 16128, 85
K_NUM = 20
NCORE, NSUB, L = 2, 16, 16
NCHUNK = 4
R = N // NCHUNK
GROUP = 64
G = R // GROUP
NEG = np.float32(-3.0e38)

_f32 = jnp.float32
_i32 = jnp.int32


def _iota16():
    return lax.iota(_i32, L)


def _find_max_and_group(gmax_ref):
    vs = [gmax_ref[pl.ds(q * L, L)] for q in range(4)]
    m = jnp.max(jnp.maximum(jnp.maximum(vs[0], vs[1]),
                            jnp.maximum(vs[2], vs[3])))
    iota = _iota16()
    big = np.int32(4 * L)
    cand = jnp.minimum(
        jnp.minimum(jnp.where(vs[0] == m, iota, big),
                    jnp.where(vs[1] == m, iota + L, big)),
        jnp.minimum(jnp.where(vs[2] == m, iota + 2 * L, big),
                    jnp.where(vs[3] == m, iota + 3 * L, big)))
    return m, jnp.min(cand)


def _remove_one_and_refresh(valbuf, gmax_ref, g, m):
    iota = _iota16()
    base = g * GROUP
    vs = [valbuf[pl.ds(base + q * L, L)] for q in range(4)]
    big = np.int32(GROUP)
    pos = jnp.min(jnp.minimum(
        jnp.minimum(jnp.where(vs[0] == m, iota, big),
                    jnp.where(vs[1] == m, iota + L, big)),
        jnp.minimum(jnp.where(vs[2] == m, iota + 2 * L, big),
                    jnp.where(vs[3] == m, iota + 3 * L, big))))
    gm = NEG
    for q in range(4):
        v2 = jnp.where(iota + q * L == pos, NEG, vs[q])
        valbuf[pl.ds(base + q * L, L)] = v2
        gm = jnp.maximum(gm, jnp.max(v2))
    plsc.store_scatter(gmax_ref, [jnp.full((L,), g, _i32)],
                       jnp.full((L,), gm, _f32), mask=iota == 0)


def _extract_topk(valbuf, gmax_ref, top_ref):
    def body(i, acc):
        m, g = _find_max_and_group(gmax_ref)
        _remove_one_and_refresh(valbuf, gmax_ref, g, m)
        plsc.store_scatter(top_ref, [jnp.full((L,), i, _i32)],
                           jnp.full((L,), m, _f32), mask=_iota16() == 0)
        return acc + m
    return lax.fori_loop(0, K_NUM, body, np.float32(0.0))


def _merge_topk_sum(merge_ref):
    iota = _iota16()
    gm0 = jnp.full((L,), NEG, _f32)
    for j in range(8):
        gm0 = jnp.where(iota == j, jnp.max(merge_ref[pl.ds(j * L, L)]), gm0)

    def body(i, carry):
        acc, gm = carry
        m = jnp.max(gm)
        g = jnp.min(jnp.where(gm == m, iota, L))
        v = merge_ref[pl.ds(g * L, L)]
        pos = jnp.min(jnp.where(v == m, iota, L))
        v2 = jnp.where(iota == pos, NEG, v)
        merge_ref[pl.ds(g * L, L)] = v2
        gm = jnp.where(iota == g, jnp.max(v2), gm)
        return (acc + m, gm)

    acc, _ = lax.fori_loop(0, K_NUM, body, (np.float32(0.0), gm0))
    return acc


def _sc_body(y_hbm, gt_hbm, out_hbm,
             cbx, cby, cbw, cbh, cbo, cbc, ioub, confb, gmax_i, gmax_c, gtv,
             top_i, top_c, shared_i, shared_c, merge_i, merge_c, outv,
             sem0, sem1, sem2, sem3, sem4, sem5):
    c = lax.axis_index("c")
    s = lax.axis_index("s")
    wid = c * NSUB + s
    b = wid // NCHUNK
    chunk = wid % NCHUNK

    row_lo = chunk * R
    sems = (sem0, sem1, sem2, sem3, sem4, sem5)
    copies = [
        pltpu.async_copy(y_hbm.at[b, cc, pl.ds(row_lo, R)], cbuf, sems[cc])
        for cc, cbuf in enumerate((cbx, cby, cbw, cbh, cbo, cbc))
    ]

    pltpu.sync_copy(gt_hbm.at[b], gtv)

    gv = gtv[pl.ds(0, L)]
    gx1, gy1, gx2, gy2, garea = gv[0], gv[1], gv[2], gv[3], gv[4]

    neg = jnp.full((L,), NEG, _f32)
    for q in range(4):
        gmax_i[pl.ds(q * L, L)] = neg
        gmax_c[pl.ds(q * L, L)] = neg
    for q in range(2):
        top_i[pl.ds(q * L, L)] = neg
        top_c[pl.ds(q * L, L)] = neg

    iota = _iota16()
    lane0 = iota == 0

    def group_body(g, _):
        gm_i = neg
        gm_c = neg
        for q in range(4):
            row0 = g * GROUP + q * L
            sl = pl.ds(row0, L)
            cx, cy, w, h, obj, cls = (cbx[sl], cby[sl], cbw[sl], cbh[sl],
                                      cbo[sl], cbc[sl])
            x1 = cx - w * 0.5
            x2 = cx + w * 0.5
            yy1 = cy - h * 0.5
            yy2 = cy + h * 0.5
            ix1 = jnp.maximum(gx1, x1)
            iy1 = jnp.maximum(gy1, yy1)
            ix2 = jnp.minimum(gx2, x2)
            iy2 = jnp.minimum(gy2, yy2)
            iw = jnp.maximum(ix2 - ix1 + 1.0, 0.0)
            ih = jnp.maximum(iy2 - iy1 + 1.0, 0.0)
            inter = iw * ih
            b2a = (x2 - x1 + 1.0) * (yy2 - yy1 + 1.0)
            iou = inter / (garea + b2a - inter + 1e-16)
            conf = obj + cls
            ioub[pl.ds(row0, L)] = iou
            confb[pl.ds(row0, L)] = conf
            gm_i = jnp.maximum(gm_i, iou)
            gm_c = jnp.maximum(gm_c, conf)
        gidx = jnp.full((L,), g, _i32)
        plsc.store_scatter(gmax_i, [gidx],
                           jnp.full((L,), jnp.max(gm_i), _f32), mask=lane0)
        plsc.store_scatter(gmax_c, [gidx],
                           jnp.full((L,), jnp.max(gm_c), _f32), mask=lane0)
        return 0

    for cp in copies:
        cp.wait()
    lax.fori_loop(0, G, group_body, 0)

    _extract_topk(ioub, gmax_i, top_i)
    _extract_topk(confb, gmax_c, top_c)

    pltpu.sync_copy(top_i, shared_i.at[pl.ds(s * 2 * L, 2 * L)])
    pltpu.sync_copy(top_c, shared_c.at[pl.ds(s * 2 * L, 2 * L)])
    plsc.subcore_barrier()

    s0 = (s // NCHUNK) * NCHUNK

    @pl.when(s % NCHUNK == 0)
    def _merge_conf():
        pltpu.sync_copy(shared_c.at[pl.ds(s0 * 2 * L, NCHUNK * 2 * L)],
                        merge_c)
        conf_sum = _merge_topk_sum(merge_c)
        outv[pl.ds(0, L)] = jnp.where(lane0, jnp.full((L,), conf_sum, _f32),
                                      jnp.zeros((L,), _f32))
        pltpu.sync_copy(outv, out_hbm.at[b, 0])

    @pl.when(s % NCHUNK == 1)
    def _merge_iou():
        pltpu.sync_copy(shared_i.at[pl.ds(s0 * 2 * L, NCHUNK * 2 * L)],
                        merge_i)
        iou_sum = _merge_topk_sum(merge_i)
        outv[pl.ds(0, L)] = jnp.where(lane0, jnp.full((L,), iou_sum, _f32),
                                      jnp.zeros((L,), _f32))
        pltpu.sync_copy(outv, out_hbm.at[b, 1])


@functools.partial(jax.jit, static_argnames=())
def _sc_call(y, gtf):
    mesh = plsc.VectorSubcoreMesh(core_axis_name="c", subcore_axis_name="s",
                                  num_cores=NCORE, num_subcores=NSUB)
    fn = pl.kernel(
        _sc_body,
        out_type=jax.ShapeDtypeStruct((B, 2, L), _f32),
        mesh=mesh,
        compiler_params=pltpu.CompilerParams(use_tc_tiling_on_sc=False,
                                             needs_layout_passes=False),
        scratch_types=[
            pltpu.VMEM((R,), _f32),
            pltpu.VMEM((R,), _f32),
            pltpu.VMEM((R,), _f32),
            pltpu.VMEM((R,), _f32),
            pltpu.VMEM((R,), _f32),
            pltpu.VMEM((R,), _f32),
            pltpu.VMEM((R,), _f32),
            pltpu.VMEM((R,), _f32),
            pltpu.VMEM((GROUP,), _f32),
            pltpu.VMEM((GROUP,), _f32),
            pltpu.VMEM((L,), _f32),
            pltpu.VMEM((2 * L,), _f32),
            pltpu.VMEM((2 * L,), _f32),
            pltpu.VMEM_SHARED((NSUB * 2 * L,), _f32),
            pltpu.VMEM_SHARED((NSUB * 2 * L,), _f32),
            pltpu.VMEM((NCHUNK * 2 * L,), _f32),
            pltpu.VMEM((NCHUNK * 2 * L,), _f32),
            pltpu.VMEM((L,), _f32),
            pltpu.SemaphoreType.DMA,
            pltpu.SemaphoreType.DMA,
            pltpu.SemaphoreType.DMA,
            pltpu.SemaphoreType.DMA,
            pltpu.SemaphoreType.DMA,
            pltpu.SemaphoreType.DMA,
        ],
    )
    return fn(y, gtf)


def kernel(YOLOoutput, conf_thres, gt_boxes):
    del conf_thres
    gt = gt_boxes.astype(_f32)
    x1, y1, x2, y2 = gt[:, 0], gt[:, 1], gt[:, 2], gt[:, 3]
    area = (x2 - x1 + 1.0) * (y2 - y1 + 1.0)
    zeros = jnp.zeros_like(x1)
    gtf = jnp.stack([x1, y1, x2, y2, area] + [zeros] * 11, axis=1)
    yt = jnp.transpose(YOLOoutput[:, :, :6], (0, 2, 1))
    out = _sc_call(yt, gtf)
    max_conf = out[:, 0, 0] / K_NUM
    max_ious = jnp.sum(out[:, 1, 0]) / K_NUM
    return (max_conf, max_ious)
